# speedup vs baseline: 2.8647x; 2.8647x over previous
import jax
import jax.numpy as jnp
from jax.experimental import pallas as pl
from jax.experimental.pallas import tpu as pltpu

BLOCK_S = 512


def _add_body(x_ref, tab_ref, out_ref):
    out_ref[0, :, :] = x_ref[0, :, :] + tab_ref[:, :]


def kernel(x, position_table):
    batch, seq_len, d_model = x.shape
    table = position_table[:seq_len]
    grid = (seq_len // BLOCK_S, batch)
    return pl.pallas_call(
        _add_body,
        grid=grid,
        in_specs=[
            pl.BlockSpec((1, BLOCK_S, d_model), lambda s, b: (b, s, 0)),
            pl.BlockSpec((BLOCK_S, d_model), lambda s, b: (s, 0)),
        ],
        out_specs=pl.BlockSpec((1, BLOCK_S, d_model), lambda s, b: (b, s, 0)),
        out_shape=jax.ShapeDtypeStruct(x.shape, x.dtype),
        compiler_params=pltpu.CompilerParams(
            dimension_semantics=("arbitrary", "arbitrary"),
        ),
    )(x, table)
